# baseline (speedup 1.0000x reference)
import functools

import jax
import jax.numpy as jnp
from jax.experimental import pallas as pl
from jax.experimental.pallas import tpu as pltpu

B = 1
L = 2048
D_MODEL = 768
D_INNER = 1536
N_EXPERTS = 8
HID = 128
D_CONV = 4
PAD_TOP = D_CONV - 1
LPAD = 2056
CBLK = 512


def _silu(x):
    return x * jax.nn.sigmoid(x)


def _moe_kernel(xpad_ref, xbf_ref, wz_ref, wh_ref, cw_ref, cb_ref, wo_ref,
                wr1_ref, wr2_ref, out_ref, wgt_ref):
    e = pl.program_id(0)
    c = pl.program_id(1)
    first = (e == 0) & (c == 0)

    xbf = xbf_ref[...]
    x2bf = xbf[PAD_TOP:PAD_TOP + L, :]

    dims = (((1,), (1,)), ((), ()))

    @pl.when(first)
    def _router():
        x2d = xpad_ref[...][PAD_TOP:PAD_TOP + L, :]
        h1 = _silu(jax.lax.dot_general(x2d, wr1_ref[...], dims,
                                       preferred_element_type=jnp.float32))
        logits = jax.lax.dot_general(h1, wr2_ref[...], dims,
                                     preferred_element_type=jnp.float32)
        iota = jax.lax.broadcasted_iota(jnp.int32, (L, N_EXPERTS), 1)
        m1 = jnp.max(logits, axis=1, keepdims=True)
        i1 = jnp.min(jnp.where(logits == m1, iota, N_EXPERTS),
                     axis=1, keepdims=True)
        masked = jnp.where(iota == i1, -jnp.inf, logits)
        m2 = jnp.max(masked, axis=1, keepdims=True)
        i2 = jnp.min(jnp.where(masked == m2, iota, N_EXPERTS),
                     axis=1, keepdims=True)
        sel = (iota == i1) | (iota == i2)
        expw = jnp.where(sel, jnp.exp(logits - m1), 0.0)
        wgt_ref[...] = expw / jnp.sum(expw, axis=1, keepdims=True)
        out_ref[...] = jnp.zeros_like(out_ref)

    wz = wz_ref[0]
    wh = wh_ref[0]
    z = jax.lax.dot_general(x2bf, wz, dims,
                            preferred_element_type=jnp.float32)
    hp = jax.lax.dot_general(xbf, wh, dims,
                             preferred_element_type=jnp.float32)

    cw = cw_ref[0]
    cv = cb_ref[0]
    for j in range(D_CONV):
        cv = cv + cw[j:j + 1, :] * hp[j:j + L, :]

    act = _silu(cv) * _silu(z)

    eio = jax.lax.broadcasted_iota(jnp.int32, (L, N_EXPERTS), 1)
    wcol = jnp.sum(jnp.where(eio == e, wgt_ref[...], 0.0),
                   axis=1, keepdims=True)
    wact = (act * wcol).astype(jnp.bfloat16)

    wo = wo_ref[0]
    part = jax.lax.dot_general(wact, wo, (((1,), (1,)), ((), ())),
                               preferred_element_type=jnp.float32)
    out_ref[...] += part


@functools.partial(jax.jit, static_argnames=())
def kernel(hidden_states, W_in, conv_w, conv_b, W_out, Wr1, Wr2):
    x2d = hidden_states.reshape(L, D_MODEL)
    xpad = jnp.pad(x2d, ((PAD_TOP, LPAD - L - PAD_TOP), (0, 0)))
    xbf = xpad.astype(jnp.bfloat16)
    Wz = W_in[:, :D_INNER, :].astype(jnp.bfloat16)
    Wh = W_in[:, D_INNER:, :].astype(jnp.bfloat16)
    Wo = W_out.astype(jnp.bfloat16)

    n_cblk = D_INNER // CBLK
    grid = (N_EXPERTS, n_cblk)

    out = pl.pallas_call(
        _moe_kernel,
        grid=grid,
        in_specs=[
            pl.BlockSpec((LPAD, D_MODEL), lambda e, c: (0, 0)),
            pl.BlockSpec((LPAD, D_MODEL), lambda e, c: (0, 0)),
            pl.BlockSpec((1, CBLK, D_MODEL), lambda e, c: (e, c, 0)),
            pl.BlockSpec((1, CBLK, D_MODEL), lambda e, c: (e, c, 0)),
            pl.BlockSpec((1, D_CONV, CBLK), lambda e, c: (e, 0, c)),
            pl.BlockSpec((1, 1, CBLK), lambda e, c: (e, 0, c)),
            pl.BlockSpec((1, D_MODEL, CBLK), lambda e, c: (e, 0, c)),
            pl.BlockSpec((HID, D_MODEL), lambda e, c: (0, 0)),
            pl.BlockSpec((N_EXPERTS, HID), lambda e, c: (0, 0)),
        ],
        out_specs=pl.BlockSpec((L, D_MODEL), lambda e, c: (0, 0)),
        out_shape=jax.ShapeDtypeStruct((L, D_MODEL), jnp.float32),
        scratch_shapes=[pltpu.VMEM((L, N_EXPERTS), jnp.float32)],
    )(xpad, xbf, Wz, Wh, conv_w.transpose(0, 2, 1),
      conv_b.reshape(N_EXPERTS, 1, D_INNER), Wo, Wr1, Wr2)
    return out.reshape(B, L, D_MODEL)


# speedup vs baseline: 1.1643x; 1.1643x over previous
import functools

import jax
import jax.numpy as jnp
from jax.experimental import pallas as pl
from jax.experimental.pallas import tpu as pltpu

B = 1
L = 2048
D_MODEL = 768
D_INNER = 1536
N_EXPERTS = 8
HID = 128
D_CONV = 4
PAD_TOP = D_CONV - 1
LPAD = 2056
CBLK = 768


def _silu(x):
    return x * jax.nn.sigmoid(x)


def _moe_kernel(xpad_ref, wz_ref, wh_ref, cw_ref, cb_ref, wo_ref,
                wr1_ref, wr2_ref, out_ref, wgt_ref):
    e = pl.program_id(0)
    c = pl.program_id(1)
    first = (e == 0) & (c == 0)

    xpad = xpad_ref[...]
    x2d = xpad[PAD_TOP:PAD_TOP + L, :]

    dims = (((1,), (1,)), ((), ()))

    @pl.when(first)
    def _router():
        h1 = _silu(jax.lax.dot_general(x2d, wr1_ref[...], dims,
                                       preferred_element_type=jnp.float32))
        logits = jax.lax.dot_general(h1, wr2_ref[...], dims,
                                     preferred_element_type=jnp.float32)
        iota = jax.lax.broadcasted_iota(jnp.int32, (L, N_EXPERTS), 1)
        m1 = jnp.max(logits, axis=1, keepdims=True)
        i1 = jnp.min(jnp.where(logits == m1, iota, N_EXPERTS),
                     axis=1, keepdims=True)
        masked = jnp.where(iota == i1, -jnp.inf, logits)
        m2 = jnp.max(masked, axis=1, keepdims=True)
        i2 = jnp.min(jnp.where(masked == m2, iota, N_EXPERTS),
                     axis=1, keepdims=True)
        sel = (iota == i1) | (iota == i2)
        expw = jnp.where(sel, jnp.exp(logits - m1), 0.0)
        wgt_ref[...] = expw / jnp.sum(expw, axis=1, keepdims=True)
        out_ref[...] = jnp.zeros_like(out_ref)

    wz = wz_ref[0]
    wh = wh_ref[0]
    z = jax.lax.dot_general(x2d, wz, dims,
                            preferred_element_type=jnp.float32)
    hp = jax.lax.dot_general(xpad, wh, dims,
                             preferred_element_type=jnp.float32)

    cw = cw_ref[0]
    cv = cb_ref[0]
    for j in range(D_CONV):
        cv = cv + cw[j:j + 1, :] * hp[j:j + L, :]

    act = _silu(cv) * _silu(z)

    eio = jax.lax.broadcasted_iota(jnp.int32, (L, N_EXPERTS), 1)
    wcol = jnp.sum(jnp.where(eio == e, wgt_ref[...], 0.0),
                   axis=1, keepdims=True)
    wact = act * wcol

    wo = wo_ref[0]
    part = jax.lax.dot_general(wact, wo, (((1,), (1,)), ((), ())),
                               preferred_element_type=jnp.float32)
    out_ref[...] += part


@functools.partial(jax.jit, static_argnames=())
def kernel(hidden_states, W_in, conv_w, conv_b, W_out, Wr1, Wr2):
    x2d = hidden_states.reshape(L, D_MODEL)
    xpad = jnp.pad(x2d, ((PAD_TOP, LPAD - L - PAD_TOP), (0, 0)))
    Wz = W_in[:, :D_INNER, :]
    Wh = W_in[:, D_INNER:, :]

    n_cblk = D_INNER // CBLK
    grid = (N_EXPERTS, n_cblk)

    out = pl.pallas_call(
        _moe_kernel,
        grid=grid,
        in_specs=[
            pl.BlockSpec((LPAD, D_MODEL), lambda e, c: (0, 0)),
            pl.BlockSpec((1, CBLK, D_MODEL), lambda e, c: (e, c, 0)),
            pl.BlockSpec((1, CBLK, D_MODEL), lambda e, c: (e, c, 0)),
            pl.BlockSpec((1, D_CONV, CBLK), lambda e, c: (e, 0, c)),
            pl.BlockSpec((1, 1, CBLK), lambda e, c: (e, 0, c)),
            pl.BlockSpec((1, D_MODEL, CBLK), lambda e, c: (e, 0, c)),
            pl.BlockSpec((HID, D_MODEL), lambda e, c: (0, 0)),
            pl.BlockSpec((N_EXPERTS, HID), lambda e, c: (0, 0)),
        ],
        out_specs=pl.BlockSpec((L, D_MODEL), lambda e, c: (0, 0)),
        out_shape=jax.ShapeDtypeStruct((L, D_MODEL), jnp.float32),
        scratch_shapes=[pltpu.VMEM((L, N_EXPERTS), jnp.float32)],
    )(xpad, Wz, Wh, conv_w.transpose(0, 2, 1),
      conv_b.reshape(N_EXPERTS, 1, D_INNER), W_out, Wr1, Wr2)
    return out.reshape(B, L, D_MODEL)
